# baseline (speedup 1.0000x reference)
import functools

import jax
import jax.numpy as jnp
from jax import lax
from jax.experimental import pallas as pl
from jax.experimental.pallas import tpu as pltpu
from jax.experimental.pallas import tpu_sc as plsc

BATCH = 4096
SEQ_LEN = 457
EMBED_DIM = 64
VOCAB = 457

NC = 2
NS = 16
NW = NC * NS
SEQB = BATCH // NW
NGRP = SEQB // 16


def _build_gather():
    mesh = plsc.VectorSubcoreMesh(core_axis_name="c", subcore_axis_name="s")

    @functools.partial(
        pl.kernel,
        out_type=jax.ShapeDtypeStruct((SEQ_LEN, EMBED_DIM, BATCH), jnp.float32),
        mesh=mesh,
        scratch_types=[
            pltpu.VMEM((VOCAB * EMBED_DIM,), jnp.float32),
            pltpu.VMEM((SEQB * SEQ_LEN,), jnp.int32),
            pltpu.VMEM((2, EMBED_DIM, SEQB), jnp.float32),
            pltpu.SemaphoreType.DMA((2,)),
        ],
        compiler_params=pltpu.CompilerParams(needs_layout_passes=False),
    )
    def gather(tbl_hbm, seq_hbm, out_hbm, tbl_v, seq_v, buf_v, wsem):
        wid = lax.axis_index("s") * NC + lax.axis_index("c")
        b0 = pl.multiple_of(wid * SEQB, SEQB)
        pltpu.sync_copy(tbl_hbm, tbl_v)
        pltpu.sync_copy(seq_hbm.at[pl.ds(b0 * SEQ_LEN, SEQB * SEQ_LEN)], seq_v)

        iot = jnp.arange(16, dtype=jnp.int32)
        gbase = [(iot + g * 16) * SEQ_LEN for g in range(NGRP)]

        def compute(s, p):
            sv = jnp.full((16,), 0, dtype=jnp.int32) + s
            for g in range(NGRP):
                idx16 = plsc.load_gather(seq_v, [gbase[g] + sv])
                addr16 = idx16 * EMBED_DIM
                for c in range(EMBED_DIM):
                    buf_v[p, c, pl.ds(g * 16, 16)] = plsc.load_gather(
                        tbl_v, [addr16 + c]
                    )

        def start_write(s, p):
            pltpu.async_copy(
                buf_v.at[p], out_hbm.at[s, :, pl.ds(b0, SEQB)], wsem.at[p]
            )

        def wait_write(s, p):
            pltpu.make_async_copy(
                buf_v.at[p], out_hbm.at[s, :, pl.ds(b0, SEQB)], wsem.at[p]
            ).wait()

        @pl.loop(0, SEQ_LEN - 1, step=2)
        def _(j):
            for p in range(2):
                s = j + p

                @pl.when(s >= 2)
                def _():
                    wait_write(s - 2, p)

                compute(s, p)
                start_write(s, p)

        wait_write(SEQ_LEN - 3, 0)
        compute(SEQ_LEN - 1, 0)
        start_write(SEQ_LEN - 1, 0)
        wait_write(SEQ_LEN - 2, 1)
        wait_write(SEQ_LEN - 1, 0)

    return gather


_gather = _build_gather()


@jax.jit
def kernel(seqs, exps, table):
    del exps
    out_t = _gather(table.reshape(-1), seqs.reshape(-1))
    return jnp.transpose(out_t, (2, 0, 1))


# speedup vs baseline: 1.8630x; 1.8630x over previous
import functools

import jax
import jax.numpy as jnp
from jax import lax
from jax.experimental import pallas as pl
from jax.experimental.pallas import tpu as pltpu
from jax.experimental.pallas import tpu_sc as plsc

BATCH = 4096
SEQ_LEN = 457
EMBED_DIM = 64
VOCAB = 457

NC = 2
NS = 16
NW = NC * NS
SEQB = BATCH // NW
NGRP = SEQB // 16


def _build_gather():
    mesh = plsc.VectorSubcoreMesh(core_axis_name="c", subcore_axis_name="s")

    @functools.partial(
        pl.kernel,
        out_type=jax.ShapeDtypeStruct((SEQ_LEN, EMBED_DIM, BATCH), jnp.float32),
        mesh=mesh,
        scratch_types=[
            pltpu.VMEM((VOCAB * EMBED_DIM,), jnp.float32),
            pltpu.VMEM((SEQB * SEQ_LEN,), jnp.int32),
            pltpu.VMEM((2, EMBED_DIM, SEQB), jnp.float32),
            pltpu.SemaphoreType.DMA((2,)),
        ],
        compiler_params=pltpu.CompilerParams(needs_layout_passes=False),
    )
    def gather(tbl_hbm, seq_hbm, out_hbm, tbl_v, seq_v, buf_v, wsem):
        wid = lax.axis_index("s") * NC + lax.axis_index("c")
        b0 = pl.multiple_of(wid * SEQB, SEQB)
        pltpu.sync_copy(tbl_hbm, tbl_v)
        pltpu.sync_copy(seq_hbm.at[pl.ds(b0 * SEQ_LEN, SEQB * SEQ_LEN)], seq_v)

        iot = jnp.arange(16, dtype=jnp.int32)
        gbase = [(iot + g * 16) * SEQ_LEN for g in range(NGRP)]

        def compute(s, p):
            sv = jnp.full((16,), 0, dtype=jnp.int32) + s
            for g in range(NGRP):
                idx16 = plsc.load_gather(seq_v, [gbase[g] + sv])
                for c in range(EMBED_DIM):
                    buf_v[p, c, pl.ds(g * 16, 16)] = plsc.load_gather(
                        tbl_v, [idx16 + c * VOCAB]
                    )

        def start_write(s, p):
            pltpu.async_copy(
                buf_v.at[p], out_hbm.at[s, :, pl.ds(b0, SEQB)], wsem.at[p]
            )

        def wait_write(s, p):
            pltpu.make_async_copy(
                buf_v.at[p], out_hbm.at[s, :, pl.ds(b0, SEQB)], wsem.at[p]
            ).wait()

        @pl.loop(0, SEQ_LEN - 1, step=2)
        def _(j):
            for p in range(2):
                s = j + p

                @pl.when(s >= 2)
                def _():
                    wait_write(s - 2, p)

                compute(s, p)
                start_write(s, p)

        wait_write(SEQ_LEN - 3, 0)
        compute(SEQ_LEN - 1, 0)
        start_write(SEQ_LEN - 1, 0)
        wait_write(SEQ_LEN - 2, 1)
        wait_write(SEQ_LEN - 1, 0)

    return gather


_gather = _build_gather()


@jax.jit
def kernel(seqs, exps, table):
    del exps
    out_t = _gather(jnp.transpose(table).reshape(-1), seqs.reshape(-1))
    return jnp.transpose(out_t, (2, 0, 1))


# speedup vs baseline: 11.9708x; 6.4257x over previous
import functools

import jax
import jax.numpy as jnp
from jax import lax
from jax.experimental import pallas as pl
from jax.experimental.pallas import tpu as pltpu
from jax.experimental.pallas import tpu_sc as plsc

BATCH = 4096
SEQ_LEN = 457
EMBED_DIM = 64
VOCAB = 457

NC = 2
NS = 16
NW = NC * NS
SEQB = BATCH // NW
NGRP = SEQB // 16


def _build_gather():
    mesh = plsc.VectorSubcoreMesh(core_axis_name="c", subcore_axis_name="s")

    @functools.partial(
        pl.kernel,
        out_type=jax.ShapeDtypeStruct((SEQ_LEN, EMBED_DIM, BATCH), jnp.float32),
        mesh=mesh,
        scratch_types=[
            pltpu.VMEM((VOCAB * EMBED_DIM,), jnp.float32),
            pltpu.VMEM((SEQB * SEQ_LEN,), jnp.int32),
            pltpu.VMEM((2, EMBED_DIM, SEQB), jnp.float32),
            pltpu.SemaphoreType.DMA((2,)),
        ],
        compiler_params=pltpu.CompilerParams(needs_layout_passes=False),
    )
    def gather(tbl_hbm, seq_hbm, out_hbm, tbl_v, seq_v, buf_v, wsem):
        wid = lax.axis_index("s") * NC + lax.axis_index("c")
        b0 = pl.multiple_of(wid * SEQB, SEQB)
        pltpu.sync_copy(tbl_hbm, tbl_v)
        pltpu.sync_copy(seq_hbm.at[pl.ds(b0 * SEQ_LEN, SEQB * SEQ_LEN)], seq_v)

        iot = jnp.arange(16, dtype=jnp.int32)
        gbase = [(iot + g * 16) * SEQ_LEN for g in range(NGRP)]

        LAG = 8

        def compute(s, p):
            sv = jnp.full((16,), 0, dtype=jnp.int32) + s
            for g in range(NGRP):
                idx16 = plsc.load_gather(seq_v, [gbase[g] + sv])
                vals = {}
                for c in range(EMBED_DIM):
                    vals[c] = plsc.load_gather(tbl_v, [idx16 + c * VOCAB])
                    if c >= LAG:
                        buf_v[p, c - LAG, pl.ds(g * 16, 16)] = vals.pop(c - LAG)
                for c in range(EMBED_DIM - LAG, EMBED_DIM):
                    buf_v[p, c, pl.ds(g * 16, 16)] = vals.pop(c)

        def start_write(s, p):
            pltpu.async_copy(
                buf_v.at[p], out_hbm.at[s, :, pl.ds(b0, SEQB)], wsem.at[p]
            )

        def wait_write(s, p):
            pltpu.make_async_copy(
                buf_v.at[p], out_hbm.at[s, :, pl.ds(b0, SEQB)], wsem.at[p]
            ).wait()

        @pl.loop(0, SEQ_LEN - 1, step=2)
        def _(j):
            for p in range(2):
                s = j + p

                @pl.when(s >= 2)
                def _():
                    wait_write(s - 2, p)

                compute(s, p)
                start_write(s, p)

        wait_write(SEQ_LEN - 3, 0)
        compute(SEQ_LEN - 1, 0)
        start_write(SEQ_LEN - 1, 0)
        wait_write(SEQ_LEN - 2, 1)
        wait_write(SEQ_LEN - 1, 0)

    return gather


_gather = _build_gather()


@jax.jit
def kernel(seqs, exps, table):
    del exps
    out_t = _gather(jnp.transpose(table).reshape(-1), seqs.reshape(-1))
    return jnp.transpose(out_t, (2, 0, 1))


# speedup vs baseline: 12.9430x; 1.0812x over previous
import functools

import jax
import jax.numpy as jnp
from jax import lax
from jax.experimental import pallas as pl
from jax.experimental.pallas import tpu as pltpu
from jax.experimental.pallas import tpu_sc as plsc

BATCH = 4096
SEQ_LEN = 457
EMBED_DIM = 64
VOCAB = 457

NC = 2
NS = 16
NW = NC * NS
SEQB = BATCH // NW
NGRP = SEQB // 16


def _build_gather():
    mesh = plsc.VectorSubcoreMesh(core_axis_name="c", subcore_axis_name="s")

    @functools.partial(
        pl.kernel,
        out_type=jax.ShapeDtypeStruct((SEQ_LEN, EMBED_DIM, BATCH), jnp.float32),
        mesh=mesh,
        scratch_types=[
            pltpu.VMEM((VOCAB * EMBED_DIM,), jnp.float32),
            pltpu.VMEM((SEQ_LEN, SEQB), jnp.int32),
            pltpu.VMEM((2, EMBED_DIM, SEQB), jnp.float32),
            pltpu.SemaphoreType.DMA((2,)),
        ],
        compiler_params=pltpu.CompilerParams(needs_layout_passes=False),
    )
    def gather(tbl_hbm, seq_hbm, out_hbm, tbl_v, seq_v, buf_v, wsem):
        wid = lax.axis_index("s") * NC + lax.axis_index("c")
        b0 = pl.multiple_of(wid * SEQB, SEQB)
        pltpu.sync_copy(tbl_hbm, tbl_v)
        pltpu.sync_copy(seq_hbm.at[:, pl.ds(b0, SEQB)], seq_v)

        iot = jnp.arange(16, dtype=jnp.int32)
        gcol = [iot + g * 16 for g in range(NGRP)]

        LAG = 8

        def compute(s, p):
            sv = jnp.full((16,), 0, dtype=jnp.int32) + s
            for g in range(NGRP):
                idx16 = plsc.load_gather(seq_v, [sv, gcol[g]])
                vals = {}
                for c in range(EMBED_DIM):
                    vals[c] = plsc.load_gather(tbl_v, [idx16 + c * VOCAB])
                    if c >= LAG:
                        buf_v[p, c - LAG, pl.ds(g * 16, 16)] = vals.pop(c - LAG)
                for c in range(EMBED_DIM - LAG, EMBED_DIM):
                    buf_v[p, c, pl.ds(g * 16, 16)] = vals.pop(c)

        def start_write(s, p):
            pltpu.async_copy(
                buf_v.at[p], out_hbm.at[s, :, pl.ds(b0, SEQB)], wsem.at[p]
            )

        def wait_write(s, p):
            pltpu.make_async_copy(
                buf_v.at[p], out_hbm.at[s, :, pl.ds(b0, SEQB)], wsem.at[p]
            ).wait()

        @pl.loop(0, SEQ_LEN - 1, step=2)
        def _(j):
            for p in range(2):
                s = j + p

                @pl.when(s >= 2)
                def _():
                    wait_write(s - 2, p)

                compute(s, p)
                start_write(s, p)

        wait_write(SEQ_LEN - 3, 0)
        compute(SEQ_LEN - 1, 0)
        start_write(SEQ_LEN - 1, 0)
        wait_write(SEQ_LEN - 2, 1)
        wait_write(SEQ_LEN - 1, 0)

    return gather


_gather = _build_gather()


@jax.jit
def kernel(seqs, exps, table):
    del exps
    out_t = _gather(jnp.transpose(table).reshape(-1), jnp.transpose(seqs))
    return jnp.transpose(out_t, (2, 0, 1))


# speedup vs baseline: 14.9850x; 1.1578x over previous
import functools

import jax
import jax.numpy as jnp
from jax import lax
from jax.experimental import pallas as pl
from jax.experimental.pallas import tpu as pltpu
from jax.experimental.pallas import tpu_sc as plsc

BATCH = 4096
SEQ_LEN = 457
EMBED_DIM = 64
VOCAB = 457

NC = 2
NS = 16
NW = NC * NS
SEQB = BATCH // NW
NGRP = SEQB // 16


def _build_gather():
    mesh = plsc.VectorSubcoreMesh(core_axis_name="c", subcore_axis_name="s")

    @functools.partial(
        pl.kernel,
        out_type=jax.ShapeDtypeStruct((SEQ_LEN, EMBED_DIM, BATCH), jnp.float32),
        mesh=mesh,
        scratch_types=[
            pltpu.VMEM((VOCAB * EMBED_DIM,), jnp.float32),
            pltpu.VMEM((SEQ_LEN, SEQB), jnp.int32),
            pltpu.VMEM((2, EMBED_DIM, SEQB), jnp.float32),
            pltpu.SemaphoreType.DMA((2,)),
        ],
        compiler_params=pltpu.CompilerParams(needs_layout_passes=False),
    )
    def gather(tbl_hbm, seq_hbm, out_hbm, tbl_v, seq_v, buf_v, wsem):
        wid = lax.axis_index("s") * NC + lax.axis_index("c")
        b0 = pl.multiple_of(wid * SEQB, SEQB)
        pltpu.sync_copy(tbl_hbm, tbl_v)
        pltpu.sync_copy(seq_hbm.at[:, pl.ds(b0, SEQB)], seq_v)

        iot = jnp.arange(16, dtype=jnp.int32)
        gcol = [iot + g * 16 for g in range(NGRP)]

        LAG = 8

        def compute(s, p):
            sv = jnp.full((16,), 0, dtype=jnp.int32) + s
            idxs = [plsc.load_gather(seq_v, [sv, gcol[g]]) for g in range(NGRP)]
            for g in range(NGRP):
                idx16 = idxs[g]
                vals = {}
                for c in range(EMBED_DIM):
                    vals[c] = plsc.load_gather(tbl_v, [idx16 + c * VOCAB])
                    if c >= LAG:
                        buf_v[p, c - LAG, pl.ds(g * 16, 16)] = vals.pop(c - LAG)
                for c in range(EMBED_DIM - LAG, EMBED_DIM):
                    buf_v[p, c, pl.ds(g * 16, 16)] = vals.pop(c)

        def start_write(s, p):
            pltpu.async_copy(
                buf_v.at[p], out_hbm.at[s, :, pl.ds(b0, SEQB)], wsem.at[p]
            )

        def wait_write(s, p):
            pltpu.make_async_copy(
                buf_v.at[p], out_hbm.at[s, :, pl.ds(b0, SEQB)], wsem.at[p]
            ).wait()

        @pl.loop(0, SEQ_LEN - 1, step=2)
        def _(j):
            for p in range(2):
                s = j + p

                @pl.when(s >= 2)
                def _():
                    wait_write(s - 2, p)

                compute(s, p)
                start_write(s, p)

        wait_write(SEQ_LEN - 3, 0)
        compute(SEQ_LEN - 1, 0)
        start_write(SEQ_LEN - 1, 0)
        wait_write(SEQ_LEN - 2, 1)
        wait_write(SEQ_LEN - 1, 0)

    return gather


_gather = _build_gather()


@jax.jit
def kernel(seqs, exps, table):
    del exps
    out_t = _gather(jnp.transpose(table).reshape(-1), jnp.transpose(seqs))
    return jnp.transpose(out_t, (2, 0, 1))
